# baseline (speedup 1.0000x reference)
import functools

import jax
import jax.numpy as jnp
from jax import lax
from jax.experimental import pallas as pl
from jax.experimental.pallas import tpu as pltpu
from jax.experimental.pallas import tpu_sc as plsc

N_ATOMS = 100000
D = 128
NSEG = 1000
SEG_PAD = 1024
CHUNK = 40
NCHUNKS = 2500
NC = 2
NS = 16
NW = NC * NS
MAXCH = 80
SEG_PER_TILE = SEG_PAD // NW
SEARCH_STEPS = 17


def _sc_segment_sums(features, idx):
    mesh = plsc.VectorSubcoreMesh(core_axis_name="c", subcore_axis_name="s")

    @functools.partial(
        pl.kernel,
        mesh=mesh,
        out_type=[
            jax.ShapeDtypeStruct((NC, SEG_PAD, D), jnp.float32),
            jax.ShapeDtypeStruct((SEG_PAD, 16), jnp.float32),
        ],
        scratch_types=[
            pltpu.VMEM((CHUNK, D), jnp.float32),
            pltpu.VMEM((CHUNK, D), jnp.float32),
            pltpu.VMEM((CHUNK,), jnp.int32),
            pltpu.VMEM((CHUNK,), jnp.int32),
            pltpu.VMEM((N_ATOMS + 16,), jnp.int32),
            pltpu.VMEM((SEG_PER_TILE, 16), jnp.float32),
            pltpu.VMEM_SHARED((SEG_PAD, D), jnp.float32),
            pltpu.SemaphoreType.DMA,
            pltpu.SemaphoreType.DMA,
            pltpu.SemaphoreType.DMA,
            pltpu.SemaphoreType.DMA,
            pltpu.SemaphoreType.DMA,
        ],
    )
    def k(feat_hbm, idx_hbm, out_feat, out_cnt, buf0, buf1, idx0, idx1,
          idx_full, cbuf, acc, sf0, sf1, si0, si1, sfull):
        c = lax.axis_index("c")
        s = lax.axis_index("s")
        wid = s * NC + c
        rows = SEG_PAD // NS
        start = 78 * wid + jnp.minimum(wid, 4)
        n = jnp.where(wid < 4, 79, 78)

        full_dma = pltpu.make_async_copy(idx_hbm, idx_full.at[pl.ds(0, N_ATOMS)],
                                         sfull)
        full_dma.start()

        def zrow(i, _):
            def zlane(j, _):
                buf0[i, pl.ds(j * 16, 16)] = jnp.zeros((16,), jnp.float32)
                return 0
            return lax.fori_loop(0, D // 16, zlane, 0)

        lax.fori_loop(0, rows, zrow, 0)
        pltpu.sync_copy(buf0.at[pl.ds(0, rows)], acc.at[pl.ds(s * rows, rows)])
        plsc.subcore_barrier()

        bufs = (buf0, buf1)
        idxs = (idx0, idx1)
        sfs = (sf0, sf1)
        sis = (si0, si1)

        def dma_pair(lc, slot):
            a = (start + lc) * CHUNK
            return (pltpu.make_async_copy(feat_hbm.at[pl.ds(a, CHUNK)],
                                          bufs[slot], sfs[slot]),
                    pltpu.make_async_copy(idx_hbm.at[pl.ds(a, CHUNK)],
                                          idxs[slot], sis[slot]))

        def fire(lc, slot):
            f, i = dma_pair(lc, slot)
            f.start()
            i.start()

        def drain(lc, slot):
            f, i = dma_pair(lc, slot)
            f.wait()
            i.wait()

        fire(0, 0)

        def body(p, _):
            c0 = 2 * p

            for q in range(2):
                cq = c0 + q

                @pl.when(cq < n)
                def _():
                    drain(cq, q)

                    @pl.when(cq + 1 < n)
                    def _():
                        fire(cq + 1, 1 - q)

                    pltpu.sync_copy(bufs[q], acc.at[idxs[q]], add=True)

            return 0

        lax.fori_loop(0, MAXCH // 2, body, 0)

        full_dma.wait()
        seg0 = wid * SEG_PER_TILE

        def lower_bound(v):
            def step(_, lohi):
                lo, hi = lohi
                mid = lax.shift_right_logical(lo + hi, 1)
                x = idx_full[pl.ds(mid, 16)][0]
                big = x >= v
                done = lo >= hi
                new_lo = jnp.where(done, lo, jnp.where(big, lo, mid + 1))
                new_hi = jnp.where(done, hi, jnp.where(big, mid, hi))
                return (new_lo, new_hi)

            lo, _ = lax.fori_loop(0, SEARCH_STEPS, step, (0, N_ATOMS))
            return lo

        def count_one(j, prev):
            nxt = lower_bound(seg0 + j + 1)
            cbuf[j] = jnp.full((16,), (nxt - prev).astype(jnp.float32),
                               jnp.float32)
            return nxt

        lax.fori_loop(0, SEG_PER_TILE, count_one, lower_bound(seg0))
        pltpu.sync_copy(cbuf, out_cnt.at[pl.ds(seg0, SEG_PER_TILE)])

        plsc.subcore_barrier()

        @pl.when(s == 0)
        def _():
            pltpu.sync_copy(acc, out_feat.at[c])

    return k(features, idx)


def _tc_finish(pf, cnt, W, b2):
    def body(pf_ref, cnt_ref, w_ref, b_ref, o_ref):
        seg = pf_ref[0] + pf_ref[1]
        r = jnp.dot(seg, w_ref[...], preferred_element_type=jnp.float32)
        r = r + cnt_ref[:, 0:1] * b_ref[...]
        o_ref[...] = r[:NSEG]

    return pl.pallas_call(
        body,
        out_shape=jax.ShapeDtypeStruct((NSEG, D), jnp.float32),
    )(pf, cnt, W, b2)


def kernel(features, structural_indices, W, b):
    pf, cnt = _sc_segment_sums(features, structural_indices)
    return _tc_finish(pf, cnt, W, b.reshape(1, D))


# speedup vs baseline: 1.4107x; 1.4107x over previous
import functools

import jax
import jax.numpy as jnp
from jax import lax
from jax.experimental import pallas as pl
from jax.experimental.pallas import tpu as pltpu
from jax.experimental.pallas import tpu_sc as plsc

N_ATOMS = 100000
D = 128
NSEG = 1000
SEG_PAD = 1024
CHUNK = 128
NFULL = 781
TAIL = N_ATOMS - NFULL * CHUNK
NUNITS = NFULL + 1
NC = 2
NS = 16
NW = NC * NS
NBUF = 4


def _sc_segment_sums(features, idx):
    mesh = plsc.VectorSubcoreMesh(core_axis_name="c", subcore_axis_name="s")

    @functools.partial(
        pl.kernel,
        mesh=mesh,
        out_type=[
            jax.ShapeDtypeStruct((NC, SEG_PAD, D), jnp.float32),
            jax.ShapeDtypeStruct((NC, SEG_PAD, D), jnp.float32),
        ],
        scratch_types=[
            pltpu.VMEM((CHUNK, D), jnp.float32),
            pltpu.VMEM((CHUNK, D), jnp.float32),
            pltpu.VMEM((CHUNK, D), jnp.float32),
            pltpu.VMEM((CHUNK, D), jnp.float32),
            pltpu.VMEM((CHUNK, D), jnp.float32),
            pltpu.VMEM((CHUNK,), jnp.int32),
            pltpu.VMEM((CHUNK,), jnp.int32),
            pltpu.VMEM((CHUNK,), jnp.int32),
            pltpu.VMEM((CHUNK,), jnp.int32),
            pltpu.VMEM((TAIL,), jnp.int32),
            pltpu.VMEM_SHARED((SEG_PAD, D), jnp.float32),
            pltpu.VMEM_SHARED((SEG_PAD, D), jnp.float32),
        ] + [pltpu.SemaphoreType.DMA] * 16,
    )
    def k(feat_hbm, idx_hbm, out_feat, out_cnt, b0, b1, b2, b3, ones_v,
          i0, i1, i2, i3, idx_t, acc, acc_cnt,
          df0, df1, df2, df3, di0, di1, di2, di3,
          sf0, sf1, sf2, sf3, so0, so1, so2, so3):
        c = lax.axis_index("c")
        s = lax.axis_index("s")
        wid = s * NC + c
        rows = SEG_PAD // NS
        start = 24 * wid + jnp.minimum(wid, 14)
        n = jnp.where(wid < 14, 25, 24)

        bufs = (b0, b1, b2, b3)
        idxs = (i0, i1, i2, i3)
        dfs = (df0, df1, df2, df3)
        dis = (di0, di1, di2, di3)
        sfs = (sf0, sf1, sf2, sf3)
        sos = (so0, so1, so2, so3)

        def zrow(i, _):
            def zlane(j, _):
                b0[i, pl.ds(j * 16, 16)] = jnp.zeros((16,), jnp.float32)
                return 0
            return lax.fori_loop(0, D // 16, zlane, 0)

        lax.fori_loop(0, rows, zrow, 0)
        pltpu.sync_copy(b0.at[pl.ds(0, rows)], acc.at[pl.ds(s * rows, rows)])
        pltpu.sync_copy(b0.at[pl.ds(0, rows)],
                        acc_cnt.at[pl.ds(s * rows, rows)])

        def orow(i, _):
            def olane(j, _):
                ones_v[i, pl.ds(j * 16, 16)] = jnp.ones((16,), jnp.float32)
                return 0
            return lax.fori_loop(0, D // 16, olane, 0)

        lax.fori_loop(0, CHUNK, orow, 0)
        plsc.subcore_barrier()

        def dma_feat(lc, q):
            a = (start + lc) * CHUNK
            return pltpu.make_async_copy(feat_hbm.at[pl.ds(a, CHUNK)],
                                         bufs[q], dfs[q])

        def dma_idx(lc, q):
            a = (start + lc) * CHUNK
            return pltpu.make_async_copy(idx_hbm.at[pl.ds(a, CHUNK)],
                                         idxs[q], dis[q])

        def scat_start(q):
            pltpu.async_copy(bufs[q], acc.at[idxs[q]], sfs[q], add=True)
            pltpu.async_copy(ones_v, acc_cnt.at[idxs[q]], sos[q], add=True)

        def scat_wait(q):
            pltpu.make_async_copy(bufs[q], acc.at[idxs[q]], sfs[q]).wait()
            pltpu.make_async_copy(ones_v, acc_cnt.at[idxs[q]], sos[q]).wait()

        for q in range(2):
            dma_feat(q, q).start()
            dma_idx(q, q).start()

        def body(p, _):
            for q in range(NBUF):
                cq = NBUF * p + q
                q2 = (q + 2) % NBUF

                @pl.when(cq < n)
                def _():
                    dma_feat(cq, q).wait()
                    dma_idx(cq, q).wait()
                    scat_start(q)

                    @pl.when(cq >= 2)
                    def _():
                        scat_wait(q2)

                    @pl.when(cq + 2 < n)
                    def _():
                        dma_feat(cq + 2, q2).start()
                        dma_idx(cq + 2, q2).start()

            return 0

        lax.fori_loop(0, 7, body, 0)

        @pl.when(n == 24)
        def _():
            scat_wait(2)
            scat_wait(3)

        @pl.when(n == 25)
        def _():
            scat_wait(3)
            scat_wait(0)

        @pl.when(wid == NW - 1)
        def _():
            a = NFULL * CHUNK
            pltpu.sync_copy(idx_hbm.at[pl.ds(a, TAIL)], idx_t)
            pltpu.sync_copy(feat_hbm.at[pl.ds(a, TAIL)],
                            b0.at[pl.ds(0, TAIL)])
            pltpu.sync_copy(b0.at[pl.ds(0, TAIL)], acc.at[idx_t], add=True)
            pltpu.sync_copy(ones_v.at[pl.ds(0, TAIL)], acc_cnt.at[idx_t],
                            add=True)

        plsc.subcore_barrier()

        @pl.when(s == 0)
        def _():
            pltpu.sync_copy(acc, out_feat.at[c])
            pltpu.sync_copy(acc_cnt, out_cnt.at[c])

    return k(features, idx)


def _tc_finish(pf, pc, W, b2):
    def body(pf_ref, pc_ref, w_ref, b_ref, o_ref):
        seg = pf_ref[0] + pf_ref[1]
        cnt = pc_ref[0] + pc_ref[1]
        r = jnp.dot(seg, w_ref[...], preferred_element_type=jnp.float32)
        r = r + cnt[:, 0:1] * b_ref[...]
        o_ref[...] = r[:NSEG]

    return pl.pallas_call(
        body,
        out_shape=jax.ShapeDtypeStruct((NSEG, D), jnp.float32),
    )(pf, pc, W, b2)


def kernel(features, structural_indices, W, b):
    pf, pc = _sc_segment_sums(features, structural_indices)
    return _tc_finish(pf, pc, W, b.reshape(1, D))


# speedup vs baseline: 1.4236x; 1.0091x over previous
import functools

import jax
import jax.numpy as jnp
from jax import lax
from jax.experimental import pallas as pl
from jax.experimental.pallas import tpu as pltpu
from jax.experimental.pallas import tpu_sc as plsc

D = 128
NSEG = 1000
SEG_PAD = 1024
CHUNK = 80
NCHUNKS = 1250
NC = 2
NS = 16
NW = NC * NS
MAXCH = 40


def _sc_segment_sums(features, idx):
    mesh = plsc.VectorSubcoreMesh(core_axis_name="c", subcore_axis_name="s")

    @functools.partial(
        pl.kernel,
        mesh=mesh,
        out_type=[
            jax.ShapeDtypeStruct((NC, SEG_PAD, D), jnp.float32),
            jax.ShapeDtypeStruct((NC, SEG_PAD, D), jnp.float32),
        ],
        scratch_types=[
            pltpu.VMEM((CHUNK, D), jnp.float32),
            pltpu.VMEM((CHUNK, D), jnp.float32),
            pltpu.VMEM((CHUNK, D), jnp.float32),
            pltpu.VMEM((CHUNK,), jnp.int32),
            pltpu.VMEM((CHUNK,), jnp.int32),
            pltpu.VMEM_SHARED((SEG_PAD, D), jnp.float32),
            pltpu.VMEM_SHARED((SEG_PAD, D), jnp.float32),
            pltpu.SemaphoreType.DMA,
            pltpu.SemaphoreType.DMA,
            pltpu.SemaphoreType.DMA,
            pltpu.SemaphoreType.DMA,
        ],
    )
    def k(feat_hbm, idx_hbm, out_feat, out_cnt, buf0, buf1, ones_v, idx0,
          idx1, acc, acc_cnt, sf0, sf1, si0, si1):
        c = lax.axis_index("c")
        s = lax.axis_index("s")
        wid = s * NC + c
        rows = SEG_PAD // NS
        start = 39 * wid + jnp.minimum(wid, 2)
        n = jnp.where(wid < 2, 40, 39)

        def zrow(i, _):
            def zlane(j, _):
                buf0[i, pl.ds(j * 16, 16)] = jnp.zeros((16,), jnp.float32)
                return 0
            return lax.fori_loop(0, D // 16, zlane, 0)

        lax.fori_loop(0, rows, zrow, 0)
        pltpu.sync_copy(buf0.at[pl.ds(0, rows)], acc.at[pl.ds(s * rows, rows)])
        pltpu.sync_copy(buf0.at[pl.ds(0, rows)],
                        acc_cnt.at[pl.ds(s * rows, rows)])

        def orow(i, _):
            def olane(j, _):
                ones_v[i, pl.ds(j * 16, 16)] = jnp.ones((16,), jnp.float32)
                return 0
            return lax.fori_loop(0, D // 16, olane, 0)

        lax.fori_loop(0, CHUNK, orow, 0)
        plsc.subcore_barrier()

        bufs = (buf0, buf1)
        idxs = (idx0, idx1)
        sfs = (sf0, sf1)
        sis = (si0, si1)

        def dma_pair(lc, slot):
            a = (start + lc) * CHUNK
            return (pltpu.make_async_copy(feat_hbm.at[pl.ds(a, CHUNK)],
                                          bufs[slot], sfs[slot]),
                    pltpu.make_async_copy(idx_hbm.at[pl.ds(a, CHUNK)],
                                          idxs[slot], sis[slot]))

        def fire(lc, slot):
            f, i = dma_pair(lc, slot)
            f.start()
            i.start()

        def drain(lc, slot):
            f, i = dma_pair(lc, slot)
            f.wait()
            i.wait()

        def scatter(slot):
            pltpu.sync_copy(bufs[slot], acc.at[idxs[slot]], add=True)
            pltpu.sync_copy(ones_v, acc_cnt.at[idxs[slot]], add=True)

        fire(0, 0)

        def body(p, _):
            c0 = 2 * p

            for q in range(2):
                cq = c0 + q

                @pl.when(cq < n)
                def _():
                    drain(cq, q)

                    @pl.when(cq + 1 < n)
                    def _():
                        fire(cq + 1, 1 - q)

                    scatter(q)

            return 0

        lax.fori_loop(0, MAXCH // 2, body, 0)
        plsc.subcore_barrier()

        @pl.when(s == 0)
        def _():
            pltpu.sync_copy(acc, out_feat.at[c])
            pltpu.sync_copy(acc_cnt, out_cnt.at[c])

    return k(features, idx)


def _tc_finish(pf, pc, W, b2):
    def body(pf_ref, pc_ref, w_ref, b_ref, o_ref):
        seg = pf_ref[0] + pf_ref[1]
        cnt = pc_ref[0] + pc_ref[1]
        r = jnp.dot(seg, w_ref[...], preferred_element_type=jnp.float32)
        r = r + cnt[:, 0:1] * b_ref[...]
        o_ref[...] = r[:NSEG]

    return pl.pallas_call(
        body,
        out_shape=jax.ShapeDtypeStruct((NSEG, D), jnp.float32),
    )(pf, pc, W, b2)


def kernel(features, structural_indices, W, b):
    pf, pc = _sc_segment_sums(features, structural_indices)
    return _tc_finish(pf, pc, W, b.reshape(1, D))
